# Initial kernel scaffold; baseline (speedup 1.0000x reference)
import jax
import jax.numpy as jnp
from jax.experimental import pallas as pl


def kernel(x, edge_index, W1, att_src1, att_dst1, b1, W2, att_src2, att_dst2, b2):
    raise NotImplementedError("write your pallas kernel here")


# speedup vs baseline: 19.6455x; 19.6455x over previous
import functools

import jax
import jax.numpy as jnp
from jax import lax
from jax.experimental import pallas as pl
from jax.experimental.pallas import tpu as pltpu
from jax.experimental.pallas import tpu_sc as plsc

_N = 10000
_E = 320000
_FIN = 128
_HEADS = 8
_NHID = 64
_HC = _HEADS * _NHID
_NCLS = 40
_D2 = 48

_NC = 2
_NS = 16
_NW = _NC * _NS
_EPW = _E // _NW
_B = 80
_NB = _EPW // _B
_RPS = 624
_TAIL = _N - _NS * _RPS

_ROWBLK = 2000
_GRID = _N // _ROWBLK


def _mesh():
    return plsc.VectorSubcoreMesh(core_axis_name="c", subcore_axis_name="s")


_SC_PARAMS = pltpu.CompilerParams(use_tc_tiling_on_sc=False)


def _t1_body(x_ref, w1_ref, a32_ref, h0, h1, h2, h3, asad_ref, adas_ref):
    h = jnp.dot(x_ref[...], w1_ref[...], preferred_element_type=jnp.float32)
    h0[...] = h[:, 0:128]
    h1[...] = h[:, 128:256]
    h2[...] = h[:, 256:384]
    h3[...] = h[:, 384:512]
    t = jnp.dot(h, a32_ref[...], preferred_element_type=jnp.float32)
    asad_ref[...] = t[:, :16]
    adas_ref[...] = t[:, 16:]


def _t1_call(x, W1, A32):
    blk = _ROWBLK
    return pl.pallas_call(
        _t1_body,
        grid=(_GRID,),
        in_specs=[
            pl.BlockSpec((blk, _FIN), lambda i: (i, 0)),
            pl.BlockSpec((_FIN, _HC), lambda i: (0, 0)),
            pl.BlockSpec((_HC, 32), lambda i: (0, 0)),
        ],
        out_specs=[pl.BlockSpec((blk, 128), lambda i: (i, 0))] * 4
        + [pl.BlockSpec((blk, 16), lambda i: (i, 0))] * 2,
        out_shape=[jax.ShapeDtypeStruct((_N, 128), jnp.float32)] * 4
        + [jax.ShapeDtypeStruct((_N, 16), jnp.float32)] * 2,
    )(x, W1, A32)


def _ew_body(tab_s, tab_d, src, dst, w_out, den_out,
             idx_s, idx_d, rs, rd, wv, zb, den_sh, sem_a, sem_b):
    c = lax.axis_index("c")
    s = lax.axis_index("s")
    wid = s * _NC + c

    def zfill(i, carry):
        zb[i] = jnp.zeros((16,), jnp.float32)
        return carry

    lax.fori_loop(0, _RPS, zfill, 0)
    pltpu.sync_copy(zb, den_sh.at[pl.ds(s * _RPS, _RPS)])

    @pl.when(s == 0)
    def _():
        pltpu.sync_copy(zb.at[pl.ds(0, _TAIL)],
                        den_sh.at[pl.ds(_NS * _RPS, _TAIL)])

    plsc.subcore_barrier()

    def batch(nb, carry):
        base = wid * _EPW + nb * _B
        pltpu.sync_copy(src.at[pl.ds(base, _B)], idx_s)
        pltpu.sync_copy(dst.at[pl.ds(base, _B)], idx_d)
        cp_a = pltpu.async_copy(tab_s.at[idx_s], rs, sem_a)
        cp_b = pltpu.async_copy(tab_d.at[idx_d], rd, sem_b)
        cp_a.wait()
        cp_b.wait()

        def ebody(b, carry2):
            v = rs[b, :] + rd[b, :]
            v = jnp.where(v >= 0.0, v, v * 0.2)
            wv[b, :] = jnp.exp(v)
            return carry2

        lax.fori_loop(0, _B, ebody, 0)
        pltpu.sync_copy(wv, w_out.at[pl.ds(base, _B)])
        pltpu.sync_copy(wv, den_sh.at[idx_d], add=True)
        return carry

    lax.fori_loop(0, _NB, batch, 0)
    plsc.subcore_barrier()
    pltpu.sync_copy(den_sh.at[pl.ds(s * _RPS, _RPS)],
                    den_out.at[c, pl.ds(s * _RPS, _RPS)])

    @pl.when(s == 0)
    def _():
        pltpu.sync_copy(den_sh.at[pl.ds(_NS * _RPS, _TAIL)],
                        den_out.at[c, pl.ds(_NS * _RPS, _TAIL)])


def _ew_call(tab_s, tab_d, src, dst):
    fn = pl.kernel(
        _ew_body,
        mesh=_mesh(),
        out_type=[
            jax.ShapeDtypeStruct((_E, 16), jnp.float32),
            jax.ShapeDtypeStruct((_NC, _N, 16), jnp.float32),
        ],
        scratch_types=[
            pltpu.VMEM((_B,), jnp.int32),
            pltpu.VMEM((_B,), jnp.int32),
            pltpu.VMEM((_B, 16), jnp.float32),
            pltpu.VMEM((_B, 16), jnp.float32),
            pltpu.VMEM((_B, 16), jnp.float32),
            pltpu.VMEM((_RPS, 16), jnp.float32),
            pltpu.VMEM_SHARED((_N, 16), jnp.float32),
            pltpu.SemaphoreType.DMA,
            pltpu.SemaphoreType.DMA,
        ],
        compiler_params=_SC_PARAMS,
    )
    return fn(tab_s, tab_d, src, dst)


def _msg_body(tabs, segs, D, args):
    nchunk = len(tabs)
    (src, dst, w_in, acc_out,
     idx_s, idx_d, rows, wv, zb, acc_sh, sem_a) = args
    c = lax.axis_index("c")
    s = lax.axis_index("s")
    wid = s * _NC + c
    nd16 = D // 16

    def zfill(i, carry):
        for j in range(nd16):
            zb[i, 16 * j:16 * (j + 1)] = jnp.zeros((16,), jnp.float32)
        return carry

    lax.fori_loop(0, 128, zfill, 0)

    for ci in range(nchunk):
        tab = tabs[ci]
        r0 = s * _RPS
        for k in range(5):
            sz = 128 if k < 4 else _RPS - 4 * 128
            pltpu.sync_copy(zb.at[pl.ds(0, sz)],
                            acc_sh.at[pl.ds(r0 + k * 128, sz)])

        @pl.when(s == 0)
        def _():
            pltpu.sync_copy(zb.at[pl.ds(0, _TAIL)],
                            acc_sh.at[pl.ds(_NS * _RPS, _TAIL)])

        plsc.subcore_barrier()

        def batch(nb, carry):
            base = wid * _EPW + nb * _B
            pltpu.sync_copy(src.at[pl.ds(base, _B)], idx_s)
            pltpu.sync_copy(dst.at[pl.ds(base, _B)], idx_d)
            cp = pltpu.async_copy(tab.at[idx_s], rows, sem_a)
            pltpu.sync_copy(w_in.at[pl.ds(base, _B)], wv)
            cp.wait()

            def ebody(b, carry2):
                wrow = wv[b, :]
                for (j0, j1, wc) in segs[ci]:
                    wsc = wrow[wc]
                    for j in range(j0 // 16, j1 // 16):
                        sl = slice(16 * j, 16 * (j + 1))
                        rows[b, sl] = rows[b, sl] * wsc
                return carry2

            lax.fori_loop(0, _B, ebody, 0)
            pltpu.sync_copy(rows, acc_sh.at[idx_d], add=True)
            return carry

        lax.fori_loop(0, _NB, batch, 0)
        plsc.subcore_barrier()
        for k in range(5):
            sz = 128 if k < 4 else _RPS - 4 * 128
            pltpu.sync_copy(acc_sh.at[pl.ds(r0 + k * 128, sz)],
                            acc_out.at[c, ci, pl.ds(r0 + k * 128, sz)])

        @pl.when(s == 0)
        def _():
            pltpu.sync_copy(acc_sh.at[pl.ds(_NS * _RPS, _TAIL)],
                            acc_out.at[c, ci, pl.ds(_NS * _RPS, _TAIL)])

        if ci + 1 < nchunk:
            plsc.subcore_barrier()


def _msg_call(tabs, segs, D, src, dst, w):
    nchunk = len(tabs)
    ntab = len(tabs)

    def body(*refs):
        tabs_r = refs[:ntab]
        rest = refs[ntab:]
        _msg_body(tabs_r, segs, D, rest)

    fn = pl.kernel(
        body,
        mesh=_mesh(),
        out_type=[
            jax.ShapeDtypeStruct((_NC, nchunk, _N, D), jnp.float32),
        ],
        scratch_types=[
            pltpu.VMEM((_B,), jnp.int32),
            pltpu.VMEM((_B,), jnp.int32),
            pltpu.VMEM((_B, D), jnp.float32),
            pltpu.VMEM((_B, 16), jnp.float32),
            pltpu.VMEM((128, D), jnp.float32),
            pltpu.VMEM_SHARED((_N, D), jnp.float32),
            pltpu.SemaphoreType.DMA,
        ],
        compiler_params=_SC_PARAMS,
    )
    out = fn(*tabs, src, dst, w)
    return out[0] if isinstance(out, (tuple, list)) else out


def _t2_body(acc_ref, den_ref, b1_ref, w2_ref, a2s_ref, a2d_ref,
             h2_ref, t2s_ref, t2d_ref):
    den = den_ref[0] + den_ref[1]
    blk = den.shape[0]
    colid = lax.broadcasted_iota(jnp.int32, (1, 128), 1)
    h2 = jnp.zeros((blk, _D2), jnp.float32)
    for c4 in range(4):
        a = acc_ref[0, c4] + acc_ref[1, c4]
        d0 = den[:, 2 * c4:2 * c4 + 1]
        d1 = den[:, 2 * c4 + 1:2 * c4 + 2]
        denf = jnp.where(colid < 64, d0, d1)
        hin = a / (denf + 1e-16) + b1_ref[:, 128 * c4:128 * (c4 + 1)]
        hin = jnp.maximum(hin, 0.0)
        h2 = h2 + jnp.dot(hin, w2_ref[128 * c4:128 * (c4 + 1), :],
                          preferred_element_type=jnp.float32)
    h2_ref[...] = h2
    s2 = jnp.sum(h2 * a2s_ref[...], axis=1, keepdims=True)
    d2 = jnp.sum(h2 * a2d_ref[...], axis=1, keepdims=True)
    t2s_ref[...] = jnp.broadcast_to(s2, (blk, 16))
    t2d_ref[...] = jnp.broadcast_to(d2, (blk, 16))


def _t2_call(acc1, den1, b1r, W2p, a2s, a2d):
    blk = _ROWBLK
    return pl.pallas_call(
        _t2_body,
        grid=(_GRID,),
        in_specs=[
            pl.BlockSpec((_NC, 4, blk, 128), lambda i: (0, 0, i, 0)),
            pl.BlockSpec((_NC, blk, 16), lambda i: (0, i, 0)),
            pl.BlockSpec((1, _HC), lambda i: (0, 0)),
            pl.BlockSpec((_HC, _D2), lambda i: (0, 0)),
            pl.BlockSpec((1, _D2), lambda i: (0, 0)),
            pl.BlockSpec((1, _D2), lambda i: (0, 0)),
        ],
        out_specs=[
            pl.BlockSpec((blk, _D2), lambda i: (i, 0)),
            pl.BlockSpec((blk, 16), lambda i: (i, 0)),
            pl.BlockSpec((blk, 16), lambda i: (i, 0)),
        ],
        out_shape=[
            jax.ShapeDtypeStruct((_N, _D2), jnp.float32),
            jax.ShapeDtypeStruct((_N, 16), jnp.float32),
            jax.ShapeDtypeStruct((_N, 16), jnp.float32),
        ],
    )(acc1, den1, b1r, W2p, a2s, a2d)


def _t3_body(acc_ref, den_ref, b2_ref, out_ref):
    den = den_ref[0, :, 0:1] + den_ref[1, :, 0:1]
    a = acc_ref[0, 0] + acc_ref[1, 0]
    a = a[:, :_NCLS] / (den + 1e-16) + b2_ref[...]
    m = jnp.max(a, axis=1, keepdims=True)
    ex = jnp.exp(a - m)
    out_ref[...] = a - m - jnp.log(jnp.sum(ex, axis=1, keepdims=True))


def _t3_call(acc2, den2, b2r):
    blk = _ROWBLK
    return pl.pallas_call(
        _t3_body,
        grid=(_GRID,),
        in_specs=[
            pl.BlockSpec((_NC, 1, blk, _D2), lambda i: (0, 0, i, 0)),
            pl.BlockSpec((_NC, blk, 16), lambda i: (0, i, 0)),
            pl.BlockSpec((1, _NCLS), lambda i: (0, 0)),
        ],
        out_specs=pl.BlockSpec((blk, _NCLS), lambda i: (i, 0)),
        out_shape=jax.ShapeDtypeStruct((_N, _NCLS), jnp.float32),
    )(acc2, den2, b2r)


def kernel(x, edge_index, W1, att_src1, att_dst1, b1,
           W2, att_src2, att_dst2, b2):
    src = edge_index[0]
    dst = edge_index[1]

    eye8 = jnp.eye(_HEADS, dtype=jnp.float32)
    a_src = (eye8[:, None, :] * att_src1[:, :, None]).reshape(_HC, _HEADS)
    a_dst = (eye8[:, None, :] * att_dst1[:, :, None]).reshape(_HC, _HEADS)
    A32 = jnp.concatenate([a_src, a_dst, a_dst, a_src], axis=1)

    h1c0, h1c1, h1c2, h1c3, asad, adas = _t1_call(x, W1, A32)

    w1e, den1 = _ew_call(asad, adas, src, dst)

    segs1 = [[(0, 64, 2 * c), (64, 128, 2 * c + 1)] for c in range(4)]
    acc1 = _msg_call([h1c0, h1c1, h1c2, h1c3], segs1, 128, src, dst, w1e)

    b1r = b1.reshape(1, _HC)
    W2p = jnp.concatenate(
        [W2, jnp.zeros((_HC, _D2 - _NCLS), jnp.float32)], axis=1)
    a2s = jnp.concatenate(
        [att_src2, jnp.zeros((1, _D2 - _NCLS), jnp.float32)], axis=1)
    a2d = jnp.concatenate(
        [att_dst2, jnp.zeros((1, _D2 - _NCLS), jnp.float32)], axis=1)

    h2pad, t2s, t2d = _t2_call(acc1, den1, b1r, W2p, a2s, a2d)

    w2e, den2 = _ew_call(t2s, t2d, src, dst)
    acc2 = _msg_call([h2pad], [[(0, _D2, 0)]], _D2, src, dst, w2e)

    b2r = b2.reshape(1, _NCLS)
    return _t3_call(acc2, den2, b2r)


# speedup vs baseline: 32.3035x; 1.6443x over previous
import jax
import jax.numpy as jnp
from jax import lax
from jax.experimental import pallas as pl
from jax.experimental.pallas import tpu as pltpu
from jax.experimental.pallas import tpu_sc as plsc

_N = 10000
_E = 320000
_FIN = 128
_HEADS = 8
_NHID = 64
_HC = _HEADS * _NHID
_NCLS = 40
_D2 = 48

_NC = 2
_NS = 16
_NW = _NC * _NS
_EPW = _E // _NW
_B = 80
_NB = _EPW // _B
_RPS = 624
_TAIL = _N - _NS * _RPS

_ROWBLK = 2000
_GRID = _N // _ROWBLK


def _mesh():
    return plsc.VectorSubcoreMesh(core_axis_name="c", subcore_axis_name="s")


_SC_PARAMS = pltpu.CompilerParams(use_tc_tiling_on_sc=False)


def _t1_body(*refs):
    x_ref, w1_ref, a32_ref = refs[:3]
    houts = refs[3:11]
    asad_ref, adas_ref = refs[11], refs[12]
    h = jnp.dot(x_ref[...], w1_ref[...], preferred_element_type=jnp.float32)
    for c in range(8):
        houts[c][...] = h[:, 64 * c:64 * (c + 1)]
    t = jnp.dot(h, a32_ref[...], preferred_element_type=jnp.float32)
    asad_ref[...] = t[:, :16]
    adas_ref[...] = t[:, 16:]


def _t1_call(x, W1, A32):
    blk = _ROWBLK
    return pl.pallas_call(
        _t1_body,
        grid=(_GRID,),
        in_specs=[
            pl.BlockSpec((blk, _FIN), lambda i: (i, 0)),
            pl.BlockSpec((_FIN, _HC), lambda i: (0, 0)),
            pl.BlockSpec((_HC, 32), lambda i: (0, 0)),
        ],
        out_specs=[pl.BlockSpec((blk, 64), lambda i: (i, 0))] * 8
        + [pl.BlockSpec((blk, 16), lambda i: (i, 0))] * 2,
        out_shape=[jax.ShapeDtypeStruct((_N, 64), jnp.float32)] * 8
        + [jax.ShapeDtypeStruct((_N, 16), jnp.float32)] * 2,
    )(x, W1, A32)


def _fused_body(nchunk, D, segs, write_w, refs):
    ntab = nchunk
    tab_s, tab_d = refs[0], refs[1]
    tabs = refs[2:2 + ntab]
    src2, dst2 = refs[2 + ntab], refs[3 + ntab]
    pos = 4 + ntab
    acc_out = refs[pos]; pos += 1
    den_out = refs[pos]; pos += 1
    if write_w:
        w_out = refs[pos]; pos += 1
    (idx_s, idx_d, rows, asv, adv, wv, zb, zden,
     acc_sh, den_sh,
     sem_h0, sem_h1, sem_a0, sem_a1, sem_b0, sem_b1,
     sem_w0, sem_w1) = refs[pos:]
    sem_h = (sem_h0, sem_h1)
    sem_a = (sem_a0, sem_a1)
    sem_b = (sem_b0, sem_b1)
    sem_w = (sem_w0, sem_w1)

    c = lax.axis_index("c")
    s = lax.axis_index("s")
    wid = s * _NC + c
    ebase = wid * _EPW
    nd16 = D // 16

    pltpu.sync_copy(src2.at[pl.ds(wid * _NB, _NB)], idx_s)
    pltpu.sync_copy(dst2.at[pl.ds(wid * _NB, _NB)], idx_d)

    def zfill(i, carry):
        for j in range(nd16):
            zb[i, 16 * j:16 * (j + 1)] = jnp.zeros((16,), jnp.float32)
        return carry

    lax.fori_loop(0, 128, zfill, 0)

    def zdfill(i, carry):
        zden[i] = jnp.zeros((16,), jnp.float32)
        return carry

    lax.fori_loop(0, _RPS, zdfill, 0)

    for ci in range(nchunk):
        tab = tabs[ci]
        r0 = s * _RPS
        for k in range(5):
            sz = 128 if k < 4 else _RPS - 4 * 128
            pltpu.sync_copy(zb.at[pl.ds(0, sz)],
                            acc_sh.at[pl.ds(r0 + k * 128, sz)])
        if ci == 0:
            pltpu.sync_copy(zden, den_sh.at[pl.ds(r0, _RPS)])

        @pl.when(s == 0)
        def _():
            pltpu.sync_copy(zb.at[pl.ds(0, _TAIL)],
                            acc_sh.at[pl.ds(_NS * _RPS, _TAIL)])
            if ci == 0:
                pltpu.sync_copy(zden.at[pl.ds(0, _TAIL)],
                                den_sh.at[pl.ds(_NS * _RPS, _TAIL)])

        plsc.subcore_barrier()

        def issue(nb, p):
            pltpu.async_copy(tab.at[idx_s.at[nb]], rows.at[p], sem_h[p])
            if ci == 0:
                pltpu.async_copy(tab_s.at[idx_s.at[nb]], asv.at[p], sem_a[p])
                pltpu.async_copy(tab_d.at[idx_d.at[nb]], adv.at[p], sem_b[p])
            elif write_w:
                pltpu.async_copy(w_out.at[pl.ds(ebase + nb * _B, _B)],
                                 wv.at[p], sem_w[p])

        def process(nb, p):
            pltpu.make_async_copy(tab.at[idx_s.at[nb]],
                                  rows.at[p], sem_h[p]).wait()
            if ci == 0:
                pltpu.make_async_copy(tab_s.at[idx_s.at[nb]],
                                      asv.at[p], sem_a[p]).wait()
                pltpu.make_async_copy(tab_d.at[idx_d.at[nb]],
                                      adv.at[p], sem_b[p]).wait()

                def ebody0(b, carry2):
                    v = asv[p, b, :] + adv[p, b, :]
                    v = jnp.where(v >= 0.0, v, v * 0.2)
                    wrow = jnp.exp(v)
                    wv[p, b, :] = wrow
                    for (j0, j1, wc) in segs[0]:
                        wsc = wrow[wc]
                        for j in range(j0 // 16, j1 // 16):
                            sl = slice(16 * j, 16 * (j + 1))
                            rows[p, b, sl] = rows[p, b, sl] * wsc
                    return carry2

                lax.fori_loop(0, _B, ebody0, 0)
                if write_w:
                    pltpu.sync_copy(wv.at[p],
                                    w_out.at[pl.ds(ebase + nb * _B, _B)])
                pltpu.sync_copy(wv.at[p], den_sh.at[idx_d.at[nb]], add=True)
            else:
                if write_w:
                    pltpu.make_async_copy(
                        w_out.at[pl.ds(ebase + nb * _B, _B)],
                        wv.at[p], sem_w[p]).wait()

                def ebody(b, carry2):
                    wrow = wv[p, b, :]
                    for (j0, j1, wc) in segs[ci]:
                        wsc = wrow[wc]
                        for j in range(j0 // 16, j1 // 16):
                            sl = slice(16 * j, 16 * (j + 1))
                            rows[p, b, sl] = rows[p, b, sl] * wsc
                    return carry2

                lax.fori_loop(0, _B, ebody, 0)
            pltpu.sync_copy(rows.at[p], acc_sh.at[idx_d.at[nb]], add=True)

        issue(0, 0)

        def lbody(k, carry):
            nb0 = 2 * k
            issue(nb0 + 1, 1)
            process(nb0, 0)
            issue(nb0 + 2, 0)
            process(nb0 + 1, 1)
            return carry

        lax.fori_loop(0, (_NB - 1) // 2, lbody, 0)
        process(_NB - 1, 0)

        plsc.subcore_barrier()
        for k in range(5):
            sz = 128 if k < 4 else _RPS - 4 * 128
            pltpu.sync_copy(acc_sh.at[pl.ds(r0 + k * 128, sz)],
                            acc_out.at[c, ci, pl.ds(r0 + k * 128, sz)])
        if ci == 0:
            pltpu.sync_copy(den_sh.at[pl.ds(r0, _RPS)],
                            den_out.at[c, pl.ds(r0, _RPS)])

        @pl.when(s == 0)
        def _():
            pltpu.sync_copy(acc_sh.at[pl.ds(_NS * _RPS, _TAIL)],
                            acc_out.at[c, ci, pl.ds(_NS * _RPS, _TAIL)])
            if ci == 0:
                pltpu.sync_copy(den_sh.at[pl.ds(_NS * _RPS, _TAIL)],
                                den_out.at[c, pl.ds(_NS * _RPS, _TAIL)])

        if ci + 1 < nchunk:
            plsc.subcore_barrier()


def _fused_call(tabs, segs, D, tab_s, tab_d, src2, dst2, write_w):
    nchunk = len(tabs)
    out_type = [
        jax.ShapeDtypeStruct((_NC, nchunk, _N, D), jnp.float32),
        jax.ShapeDtypeStruct((_NC, _N, 16), jnp.float32),
    ]
    if write_w:
        out_type.append(jax.ShapeDtypeStruct((_E, 16), jnp.float32))

    def body(*refs):
        _fused_body(nchunk, D, segs, write_w, refs)

    fn = pl.kernel(
        body,
        mesh=_mesh(),
        out_type=out_type,
        scratch_types=[
            pltpu.VMEM((_NB, _B), jnp.int32),
            pltpu.VMEM((_NB, _B), jnp.int32),
            pltpu.VMEM((2, _B, D), jnp.float32),
            pltpu.VMEM((2, _B, 16), jnp.float32),
            pltpu.VMEM((2, _B, 16), jnp.float32),
            pltpu.VMEM((2, _B, 16), jnp.float32),
            pltpu.VMEM((128, D), jnp.float32),
            pltpu.VMEM((_RPS, 16), jnp.float32),
            pltpu.VMEM_SHARED((_N, D), jnp.float32),
            pltpu.VMEM_SHARED((_N, 16), jnp.float32),
        ] + [pltpu.SemaphoreType.DMA] * 8,
        compiler_params=_SC_PARAMS,
    )
    return fn(tab_s, tab_d, *tabs, src2, dst2)


def _t2_body(acc_ref, den_ref, b1_ref, w2_ref, a2s_ref, a2d_ref,
             h2_ref, t2s_ref, t2d_ref):
    den = den_ref[0] + den_ref[1]
    blk = den.shape[0]
    h2 = jnp.zeros((blk, _D2), jnp.float32)
    for c8 in range(8):
        a = acc_ref[0, c8] + acc_ref[1, c8]
        denf = den[:, c8:c8 + 1]
        hin = a / (denf + 1e-16) + b1_ref[:, 64 * c8:64 * (c8 + 1)]
        hin = jnp.maximum(hin, 0.0)
        h2 = h2 + jnp.dot(hin, w2_ref[64 * c8:64 * (c8 + 1), :],
                          preferred_element_type=jnp.float32)
    h2_ref[...] = h2
    s2 = jnp.sum(h2 * a2s_ref[...], axis=1, keepdims=True)
    d2 = jnp.sum(h2 * a2d_ref[...], axis=1, keepdims=True)
    t2s_ref[...] = jnp.broadcast_to(s2, (blk, 16))
    t2d_ref[...] = jnp.broadcast_to(d2, (blk, 16))


def _t2_call(acc1, den1, b1r, W2p, a2s, a2d):
    blk = _ROWBLK
    return pl.pallas_call(
        _t2_body,
        grid=(_GRID,),
        in_specs=[
            pl.BlockSpec((_NC, 8, blk, 64), lambda i: (0, 0, i, 0)),
            pl.BlockSpec((_NC, blk, 16), lambda i: (0, i, 0)),
            pl.BlockSpec((1, _HC), lambda i: (0, 0)),
            pl.BlockSpec((_HC, _D2), lambda i: (0, 0)),
            pl.BlockSpec((1, _D2), lambda i: (0, 0)),
            pl.BlockSpec((1, _D2), lambda i: (0, 0)),
        ],
        out_specs=[
            pl.BlockSpec((blk, _D2), lambda i: (i, 0)),
            pl.BlockSpec((blk, 16), lambda i: (i, 0)),
            pl.BlockSpec((blk, 16), lambda i: (i, 0)),
        ],
        out_shape=[
            jax.ShapeDtypeStruct((_N, _D2), jnp.float32),
            jax.ShapeDtypeStruct((_N, 16), jnp.float32),
            jax.ShapeDtypeStruct((_N, 16), jnp.float32),
        ],
    )(acc1, den1, b1r, W2p, a2s, a2d)


def _t3_body(acc_ref, den_ref, b2_ref, out_ref):
    den = den_ref[0, :, 0:1] + den_ref[1, :, 0:1]
    a = acc_ref[0, 0] + acc_ref[1, 0]
    a = a[:, :_NCLS] / (den + 1e-16) + b2_ref[...]
    m = jnp.max(a, axis=1, keepdims=True)
    ex = jnp.exp(a - m)
    out_ref[...] = a - m - jnp.log(jnp.sum(ex, axis=1, keepdims=True))


def _t3_call(acc2, den2, b2r):
    blk = _ROWBLK
    return pl.pallas_call(
        _t3_body,
        grid=(_GRID,),
        in_specs=[
            pl.BlockSpec((_NC, 1, blk, _D2), lambda i: (0, 0, i, 0)),
            pl.BlockSpec((_NC, blk, 16), lambda i: (0, i, 0)),
            pl.BlockSpec((1, _NCLS), lambda i: (0, 0)),
        ],
        out_specs=pl.BlockSpec((blk, _NCLS), lambda i: (i, 0)),
        out_shape=jax.ShapeDtypeStruct((_N, _NCLS), jnp.float32),
    )(acc2, den2, b2r)


def kernel(x, edge_index, W1, att_src1, att_dst1, b1,
           W2, att_src2, att_dst2, b2):
    src2 = edge_index[0].reshape(_NW * _NB, _B)
    dst2 = edge_index[1].reshape(_NW * _NB, _B)

    eye8 = jnp.eye(_HEADS, dtype=jnp.float32)
    a_src = (eye8[:, None, :] * att_src1[:, :, None]).reshape(_HC, _HEADS)
    a_dst = (eye8[:, None, :] * att_dst1[:, :, None]).reshape(_HC, _HEADS)
    A32 = jnp.concatenate([a_src, a_dst, a_dst, a_src], axis=1)

    t1_outs = _t1_call(x, W1, A32)
    h1cs, asad, adas = list(t1_outs[:8]), t1_outs[8], t1_outs[9]

    segs1 = [[(0, 64, c)] for c in range(8)]
    acc1, den1, _ = _fused_call(h1cs, segs1, 64,
                                asad, adas, src2, dst2, write_w=True)

    b1r = b1.reshape(1, _HC)
    W2p = jnp.concatenate(
        [W2, jnp.zeros((_HC, _D2 - _NCLS), jnp.float32)], axis=1)
    a2s = jnp.concatenate(
        [att_src2, jnp.zeros((1, _D2 - _NCLS), jnp.float32)], axis=1)
    a2d = jnp.concatenate(
        [att_dst2, jnp.zeros((1, _D2 - _NCLS), jnp.float32)], axis=1)

    h2pad, t2s, t2d = _t2_call(acc1, den1, b1r, W2p, a2s, a2d)

    acc2, den2 = _fused_call([h2pad], [[(0, _D2, 0)]], _D2,
                             t2s, t2d, src2, dst2, write_w=False)

    b2r = b2.reshape(1, _NCLS)
    return _t3_call(acc2, den2, b2r)


# speedup vs baseline: 37.2101x; 1.1519x over previous
import jax
import jax.numpy as jnp
from jax import lax
from jax.experimental import pallas as pl
from jax.experimental.pallas import tpu as pltpu
from jax.experimental.pallas import tpu_sc as plsc

_N = 10000
_E = 320000
_FIN = 128
_HEADS = 8
_NHID = 64
_HC = _HEADS * _NHID
_NCLS = 40
_D2 = 48

_NC = 2
_NS = 16
_NW = _NC * _NS
_EPW = _E // _NW
_B = 80
_NB = _EPW // _B
_RPS = 624
_TAIL = _N - _NS * _RPS

_ROWBLK = 2000
_GRID = _N // _ROWBLK


def _mesh():
    return plsc.VectorSubcoreMesh(core_axis_name="c", subcore_axis_name="s")


_SC_PARAMS = pltpu.CompilerParams(use_tc_tiling_on_sc=False)


def _t1_body(*refs):
    x_ref, w1_ref, a32_ref = refs[:3]
    houts = refs[3:11]
    asad_ref, adas_ref = refs[11], refs[12]
    h = jnp.dot(x_ref[...], w1_ref[...], preferred_element_type=jnp.float32)
    for c in range(8):
        houts[c][...] = h[:, 64 * c:64 * (c + 1)]
    t = jnp.dot(h, a32_ref[...], preferred_element_type=jnp.float32)
    asad_ref[...] = t[:, :16]
    adas_ref[...] = t[:, 16:]


def _t1_call(x, W1, A32):
    blk = _ROWBLK
    return pl.pallas_call(
        _t1_body,
        grid=(_GRID,),
        in_specs=[
            pl.BlockSpec((blk, _FIN), lambda i: (i, 0)),
            pl.BlockSpec((_FIN, _HC), lambda i: (0, 0)),
            pl.BlockSpec((_HC, 32), lambda i: (0, 0)),
        ],
        out_specs=[pl.BlockSpec((blk, 64), lambda i: (i, 0))] * 8
        + [pl.BlockSpec((blk, 16), lambda i: (i, 0))] * 2,
        out_shape=[jax.ShapeDtypeStruct((_N, 64), jnp.float32)] * 8
        + [jax.ShapeDtypeStruct((_N, 16), jnp.float32)] * 2,
    )(x, W1, A32)


def _fused_body(nchunk, D, segs, write_w, refs):
    ntab = nchunk
    tab_s, tab_d = refs[0], refs[1]
    tabs = refs[2:2 + ntab]
    src2, dst2 = refs[2 + ntab], refs[3 + ntab]
    pos = 4 + ntab
    acc_out = refs[pos]; pos += 1
    den_out = refs[pos]; pos += 1
    if write_w:
        w_out = refs[pos]; pos += 1
    (idx_s, idx_d, rows, asv, adv, wv, zb, zden,
     acc_sh, den_sh) = refs[pos:pos + 10]
    sems = refs[pos + 10:]
    sem_h, sem_a, sem_b, sem_w, sem_s, sem_dw, sem_ww = (
        sems[0:3], sems[3:6], sems[6:9], sems[9:12], sems[12:15],
        sems[15:18], sems[18:21])

    c = lax.axis_index("c")
    s = lax.axis_index("s")
    wid = s * _NC + c
    ebase = wid * _EPW
    nd16 = D // 16

    pltpu.sync_copy(src2.at[pl.ds(wid * _NB, _NB)], idx_s)
    pltpu.sync_copy(dst2.at[pl.ds(wid * _NB, _NB)], idx_d)

    def zfill(i, carry):
        for j in range(nd16):
            zb[i, 16 * j:16 * (j + 1)] = jnp.zeros((16,), jnp.float32)
        return carry

    lax.fori_loop(0, 128, zfill, 0)

    def zdfill(i, carry):
        zden[i] = jnp.zeros((16,), jnp.float32)
        return carry

    lax.fori_loop(0, _RPS, zdfill, 0)

    for ci in range(nchunk):
        tab = tabs[ci]
        r0 = s * _RPS
        for k in range(5):
            sz = 128 if k < 4 else _RPS - 4 * 128
            pltpu.sync_copy(zb.at[pl.ds(0, sz)],
                            acc_sh.at[pl.ds(r0 + k * 128, sz)])
        if ci == 0:
            pltpu.sync_copy(zden, den_sh.at[pl.ds(r0, _RPS)])

        @pl.when(s == 0)
        def _():
            pltpu.sync_copy(zb.at[pl.ds(0, _TAIL)],
                            acc_sh.at[pl.ds(_NS * _RPS, _TAIL)])
            if ci == 0:
                pltpu.sync_copy(zden.at[pl.ds(0, _TAIL)],
                                den_sh.at[pl.ds(_NS * _RPS, _TAIL)])

        plsc.subcore_barrier()

        def issue(nb, q, wait_scatter):
            if wait_scatter:
                pltpu.make_async_copy(rows.at[q], acc_sh.at[idx_d.at[0]],
                                      sem_s[q]).wait()
            pltpu.async_copy(tab.at[idx_s.at[nb]], rows.at[q], sem_h[q])
            if ci == 0:
                pltpu.async_copy(tab_s.at[idx_s.at[nb]], asv.at[q], sem_a[q])
                pltpu.async_copy(tab_d.at[idx_d.at[nb]], adv.at[q], sem_b[q])
            elif write_w:
                pltpu.async_copy(w_out.at[pl.ds(ebase + nb * _B, _B)],
                                 wv.at[q], sem_w[q])

        def process(nb, q, first):
            pltpu.make_async_copy(tab.at[idx_s.at[nb]],
                                  rows.at[q], sem_h[q]).wait()
            if ci == 0:
                pltpu.make_async_copy(tab_s.at[idx_s.at[nb]],
                                      asv.at[q], sem_a[q]).wait()
                pltpu.make_async_copy(tab_d.at[idx_d.at[nb]],
                                      adv.at[q], sem_b[q]).wait()
                if not first:
                    pltpu.make_async_copy(wv.at[q], den_sh.at[idx_d.at[0]],
                                          sem_dw[q]).wait()
                    if write_w:
                        pltpu.make_async_copy(
                            wv.at[q], w_out.at[pl.ds(ebase, _B)],
                            sem_ww[q]).wait()

                def ebody0(b, carry2):
                    v = asv[q, b, :] + adv[q, b, :]
                    v = jnp.where(v >= 0.0, v, v * 0.2)
                    wrow = jnp.exp(v)
                    wv[q, b, :] = wrow
                    for (j0, j1, wc) in segs[0]:
                        wsc = wrow[wc]
                        for j in range(j0 // 16, j1 // 16):
                            sl = slice(16 * j, 16 * (j + 1))
                            rows[q, b, sl] = rows[q, b, sl] * wsc
                    return carry2

                lax.fori_loop(0, _B, ebody0, 0)
                if write_w:
                    pltpu.async_copy(wv.at[q],
                                     w_out.at[pl.ds(ebase + nb * _B, _B)],
                                     sem_ww[q])
                pltpu.async_copy(wv.at[q], den_sh.at[idx_d.at[nb]],
                                 sem_dw[q], add=True)
            else:
                if write_w:
                    pltpu.make_async_copy(
                        w_out.at[pl.ds(ebase + nb * _B, _B)],
                        wv.at[q], sem_w[q]).wait()

                def ebody(b, carry2):
                    wrow = wv[q, b, :]
                    for (j0, j1, wc) in segs[ci]:
                        wsc = wrow[wc]
                        for j in range(j0 // 16, j1 // 16):
                            sl = slice(16 * j, 16 * (j + 1))
                            rows[q, b, sl] = rows[q, b, sl] * wsc
                    return carry2

                lax.fori_loop(0, _B, ebody, 0)
            pltpu.async_copy(rows.at[q], acc_sh.at[idx_d.at[nb]],
                             sem_s[q], add=True)

        issue(0, 0, False)
        issue(1, 1, False)
        process(0, 0, True)
        issue(2, 2, False)
        process(1, 1, True)
        issue(3, 0, True)
        process(2, 2, True)
        issue(4, 1, True)

        def lbody(k, carry):
            nb0 = 3 * k
            process(nb0, 0, False)
            issue(nb0 + 2, 2, True)
            process(nb0 + 1, 1, False)
            issue(nb0 + 3, 0, True)
            process(nb0 + 2, 2, False)
            issue(nb0 + 4, 1, True)
            return carry

        lax.fori_loop(1, 41, lbody, 0)
        process(123, 0, False)
        process(124, 1, False)

        for q in range(3):
            pltpu.make_async_copy(rows.at[q], acc_sh.at[idx_d.at[0]],
                                  sem_s[q]).wait()
            if ci == 0:
                pltpu.make_async_copy(wv.at[q], den_sh.at[idx_d.at[0]],
                                      sem_dw[q]).wait()
                if write_w:
                    pltpu.make_async_copy(wv.at[q],
                                          w_out.at[pl.ds(ebase, _B)],
                                          sem_ww[q]).wait()

        plsc.subcore_barrier()
        for k in range(5):
            sz = 128 if k < 4 else _RPS - 4 * 128
            pltpu.sync_copy(acc_sh.at[pl.ds(r0 + k * 128, sz)],
                            acc_out.at[c, ci, pl.ds(r0 + k * 128, sz)])
        if ci == 0:
            pltpu.sync_copy(den_sh.at[pl.ds(r0, _RPS)],
                            den_out.at[c, pl.ds(r0, _RPS)])

        @pl.when(s == 0)
        def _():
            pltpu.sync_copy(acc_sh.at[pl.ds(_NS * _RPS, _TAIL)],
                            acc_out.at[c, ci, pl.ds(_NS * _RPS, _TAIL)])
            if ci == 0:
                pltpu.sync_copy(den_sh.at[pl.ds(_NS * _RPS, _TAIL)],
                                den_out.at[c, pl.ds(_NS * _RPS, _TAIL)])

        if ci + 1 < nchunk:
            plsc.subcore_barrier()


def _fused_call(tabs, segs, D, tab_s, tab_d, src2, dst2, write_w):
    nchunk = len(tabs)
    out_type = [
        jax.ShapeDtypeStruct((_NC, nchunk, _N, D), jnp.float32),
        jax.ShapeDtypeStruct((_NC, _N, 16), jnp.float32),
    ]
    if write_w:
        out_type.append(jax.ShapeDtypeStruct((_E, 16), jnp.float32))

    def body(*refs):
        _fused_body(nchunk, D, segs, write_w, refs)

    fn = pl.kernel(
        body,
        mesh=_mesh(),
        out_type=out_type,
        scratch_types=[
            pltpu.VMEM((_NB, _B), jnp.int32),
            pltpu.VMEM((_NB, _B), jnp.int32),
            pltpu.VMEM((3, _B, D), jnp.float32),
            pltpu.VMEM((3, _B, 16), jnp.float32),
            pltpu.VMEM((3, _B, 16), jnp.float32),
            pltpu.VMEM((3, _B, 16), jnp.float32),
            pltpu.VMEM((128, D), jnp.float32),
            pltpu.VMEM((_RPS, 16), jnp.float32),
            pltpu.VMEM_SHARED((_N, D), jnp.float32),
            pltpu.VMEM_SHARED((_N, 16), jnp.float32),
        ] + [pltpu.SemaphoreType.DMA] * 21,
        compiler_params=_SC_PARAMS,
    )
    return fn(tab_s, tab_d, *tabs, src2, dst2)


def _t2_body(acc_ref, den_ref, b1_ref, w2_ref, a2s_ref, a2d_ref,
             h2_ref, t2s_ref, t2d_ref):
    den = den_ref[0] + den_ref[1]
    blk = den.shape[0]
    h2 = jnp.zeros((blk, _D2), jnp.float32)
    for c8 in range(8):
        a = acc_ref[0, c8] + acc_ref[1, c8]
        denf = den[:, c8:c8 + 1]
        hin = a / (denf + 1e-16) + b1_ref[:, 64 * c8:64 * (c8 + 1)]
        hin = jnp.maximum(hin, 0.0)
        h2 = h2 + jnp.dot(hin, w2_ref[64 * c8:64 * (c8 + 1), :],
                          preferred_element_type=jnp.float32)
    h2_ref[...] = h2
    s2 = jnp.sum(h2 * a2s_ref[...], axis=1, keepdims=True)
    d2 = jnp.sum(h2 * a2d_ref[...], axis=1, keepdims=True)
    t2s_ref[...] = jnp.broadcast_to(s2, (blk, 16))
    t2d_ref[...] = jnp.broadcast_to(d2, (blk, 16))


def _t2_call(acc1, den1, b1r, W2p, a2s, a2d):
    blk = _ROWBLK
    return pl.pallas_call(
        _t2_body,
        grid=(_GRID,),
        in_specs=[
            pl.BlockSpec((_NC, 8, blk, 64), lambda i: (0, 0, i, 0)),
            pl.BlockSpec((_NC, blk, 16), lambda i: (0, i, 0)),
            pl.BlockSpec((1, _HC), lambda i: (0, 0)),
            pl.BlockSpec((_HC, _D2), lambda i: (0, 0)),
            pl.BlockSpec((1, _D2), lambda i: (0, 0)),
            pl.BlockSpec((1, _D2), lambda i: (0, 0)),
        ],
        out_specs=[
            pl.BlockSpec((blk, _D2), lambda i: (i, 0)),
            pl.BlockSpec((blk, 16), lambda i: (i, 0)),
            pl.BlockSpec((blk, 16), lambda i: (i, 0)),
        ],
        out_shape=[
            jax.ShapeDtypeStruct((_N, _D2), jnp.float32),
            jax.ShapeDtypeStruct((_N, 16), jnp.float32),
            jax.ShapeDtypeStruct((_N, 16), jnp.float32),
        ],
    )(acc1, den1, b1r, W2p, a2s, a2d)


def _t3_body(acc_ref, den_ref, b2_ref, out_ref):
    den = den_ref[0, :, 0:1] + den_ref[1, :, 0:1]
    a = acc_ref[0, 0] + acc_ref[1, 0]
    a = a[:, :_NCLS] / (den + 1e-16) + b2_ref[...]
    m = jnp.max(a, axis=1, keepdims=True)
    ex = jnp.exp(a - m)
    out_ref[...] = a - m - jnp.log(jnp.sum(ex, axis=1, keepdims=True))


def _t3_call(acc2, den2, b2r):
    blk = _ROWBLK
    return pl.pallas_call(
        _t3_body,
        grid=(_GRID,),
        in_specs=[
            pl.BlockSpec((_NC, 1, blk, _D2), lambda i: (0, 0, i, 0)),
            pl.BlockSpec((_NC, blk, 16), lambda i: (0, i, 0)),
            pl.BlockSpec((1, _NCLS), lambda i: (0, 0)),
        ],
        out_specs=pl.BlockSpec((blk, _NCLS), lambda i: (i, 0)),
        out_shape=jax.ShapeDtypeStruct((_N, _NCLS), jnp.float32),
    )(acc2, den2, b2r)


def kernel(x, edge_index, W1, att_src1, att_dst1, b1,
           W2, att_src2, att_dst2, b2):
    src2 = edge_index[0].reshape(_NW * _NB, _B)
    dst2 = edge_index[1].reshape(_NW * _NB, _B)

    eye8 = jnp.eye(_HEADS, dtype=jnp.float32)
    a_src = (eye8[:, None, :] * att_src1[:, :, None]).reshape(_HC, _HEADS)
    a_dst = (eye8[:, None, :] * att_dst1[:, :, None]).reshape(_HC, _HEADS)
    A32 = jnp.concatenate([a_src, a_dst, a_dst, a_src], axis=1)

    t1_outs = _t1_call(x, W1, A32)
    h1cs, asad, adas = list(t1_outs[:8]), t1_outs[8], t1_outs[9]

    segs1 = [[(0, 64, c)] for c in range(8)]
    acc1, den1, _ = _fused_call(h1cs, segs1, 64,
                                asad, adas, src2, dst2, write_w=True)

    b1r = b1.reshape(1, _HC)
    W2p = jnp.concatenate(
        [W2, jnp.zeros((_HC, _D2 - _NCLS), jnp.float32)], axis=1)
    a2s = jnp.concatenate(
        [att_src2, jnp.zeros((1, _D2 - _NCLS), jnp.float32)], axis=1)
    a2d = jnp.concatenate(
        [att_dst2, jnp.zeros((1, _D2 - _NCLS), jnp.float32)], axis=1)

    h2pad, t2s, t2d = _t2_call(acc1, den1, b1r, W2p, a2s, a2d)

    acc2, den2 = _fused_call([h2pad], [[(0, _D2, 0)]], _D2,
                             t2s, t2d, src2, dst2, write_w=False)

    b2r = b2.reshape(1, _NCLS)
    return _t3_call(acc2, den2, b2r)


# speedup vs baseline: 37.3034x; 1.0025x over previous
import jax
import jax.numpy as jnp
from jax import lax
from jax.experimental import pallas as pl
from jax.experimental.pallas import tpu as pltpu
from jax.experimental.pallas import tpu_sc as plsc

_N = 10000
_E = 320000
_FIN = 128
_HEADS = 8
_NHID = 64
_HC = _HEADS * _NHID
_NCLS = 40
_D2 = 48

_NC = 2
_NS = 16
_NW = _NC * _NS
_EPW = _E // _NW
_B = 80
_NB = _EPW // _B
_RPS = 624
_TAIL = _N - _NS * _RPS

_ROWBLK = 2000
_GRID = _N // _ROWBLK


def _mesh():
    return plsc.VectorSubcoreMesh(core_axis_name="c", subcore_axis_name="s")


_SC_PARAMS = pltpu.CompilerParams(use_tc_tiling_on_sc=False)


def _t1_body(*refs):
    x_ref, w1_ref, a32_ref = refs[:3]
    houts = refs[3:11]
    asad_ref, adas_ref = refs[11], refs[12]
    h = jnp.dot(x_ref[...], w1_ref[...], preferred_element_type=jnp.float32)
    for c in range(8):
        houts[c][...] = h[:, 64 * c:64 * (c + 1)]
    t = jnp.dot(h, a32_ref[...], preferred_element_type=jnp.float32)
    asad_ref[...] = t[:, :16]
    adas_ref[...] = t[:, 16:]


def _t1_call(x, W1, A32):
    blk = _ROWBLK
    return pl.pallas_call(
        _t1_body,
        grid=(_GRID,),
        in_specs=[
            pl.BlockSpec((blk, _FIN), lambda i: (i, 0)),
            pl.BlockSpec((_FIN, _HC), lambda i: (0, 0)),
            pl.BlockSpec((_HC, 32), lambda i: (0, 0)),
        ],
        out_specs=[pl.BlockSpec((blk, 64), lambda i: (i, 0))] * 8
        + [pl.BlockSpec((blk, 16), lambda i: (i, 0))] * 2,
        out_shape=[jax.ShapeDtypeStruct((_N, 64), jnp.float32)] * 8
        + [jax.ShapeDtypeStruct((_N, 16), jnp.float32)] * 2,
    )(x, W1, A32)


def _fused_body(nchunk, D, segs, write_w, refs):
    ntab = nchunk
    tab_s, tab_d = refs[0], refs[1]
    tabs = refs[2:2 + ntab]
    src2, dst2 = refs[2 + ntab], refs[3 + ntab]
    pos = 4 + ntab
    acc_out = refs[pos]; pos += 1
    den_out = refs[pos]; pos += 1
    if write_w:
        w_out = refs[pos]; pos += 1
    (idx_s, idx_d, rows, asv, adv, wv, zb, zden,
     acc_sh, den_sh) = refs[pos:pos + 10]
    sems = refs[pos + 10:]
    sem_h, sem_a, sem_b, sem_w, sem_s, sem_dw, sem_ww = (
        sems[0:3], sems[3:6], sems[6:9], sems[9:12], sems[12:15],
        sems[15:18], sems[18:21])

    c = lax.axis_index("c")
    s = lax.axis_index("s")
    wid = s * _NC + c
    ebase = wid * _EPW
    nd16 = D // 16

    pltpu.sync_copy(src2.at[pl.ds(wid * _NB, _NB)], idx_s)
    pltpu.sync_copy(dst2.at[pl.ds(wid * _NB, _NB)], idx_d)

    def zfill(i, carry):
        for j in range(nd16):
            zb[i, 16 * j:16 * (j + 1)] = jnp.zeros((16,), jnp.float32)
        return carry

    lax.fori_loop(0, 128, zfill, 0)

    def zdfill(i, carry):
        zden[i] = jnp.zeros((16,), jnp.float32)
        return carry

    lax.fori_loop(0, _RPS, zdfill, 0)

    for ci in range(nchunk):
        tab = tabs[ci]
        r0 = s * _RPS
        for k in range(5):
            sz = 128 if k < 4 else _RPS - 4 * 128
            pltpu.sync_copy(zb.at[pl.ds(0, sz)],
                            acc_sh.at[pl.ds(r0 + k * 128, sz)])
        if ci == 0:
            pltpu.sync_copy(zden, den_sh.at[pl.ds(r0, _RPS)])

        @pl.when(s == 0)
        def _():
            pltpu.sync_copy(zb.at[pl.ds(0, _TAIL)],
                            acc_sh.at[pl.ds(_NS * _RPS, _TAIL)])
            if ci == 0:
                pltpu.sync_copy(zden.at[pl.ds(0, _TAIL)],
                                den_sh.at[pl.ds(_NS * _RPS, _TAIL)])

        plsc.subcore_barrier()

        def issue(nb, q, wait_scatter):
            if wait_scatter:
                pltpu.make_async_copy(rows.at[q], acc_sh.at[idx_d.at[0]],
                                      sem_s[q]).wait()
            pltpu.async_copy(tab.at[idx_s.at[nb]], rows.at[q], sem_h[q])
            if ci == 0:
                pltpu.async_copy(tab_s.at[idx_s.at[nb]], asv.at[q], sem_a[q])
                pltpu.async_copy(tab_d.at[idx_d.at[nb]], adv.at[q], sem_b[q])
            elif write_w:
                pltpu.async_copy(w_out.at[pl.ds(ebase + nb * _B, _B)],
                                 wv.at[q], sem_w[q])

        def process(nb, q, first):
            pltpu.make_async_copy(tab.at[idx_s.at[nb]],
                                  rows.at[q], sem_h[q]).wait()
            if ci == 0:
                pltpu.make_async_copy(tab_s.at[idx_s.at[nb]],
                                      asv.at[q], sem_a[q]).wait()
                pltpu.make_async_copy(tab_d.at[idx_d.at[nb]],
                                      adv.at[q], sem_b[q]).wait()
                if not first:
                    pltpu.make_async_copy(wv.at[q], den_sh.at[idx_d.at[0]],
                                          sem_dw[q]).wait()
                    if write_w:
                        pltpu.make_async_copy(
                            wv.at[q], w_out.at[pl.ds(ebase, _B)],
                            sem_ww[q]).wait()

                def ebody0(b, carry2):
                    v = asv[q, b, :] + adv[q, b, :]
                    v = jnp.where(v >= 0.0, v, v * 0.2)
                    wrow = jnp.exp(v)
                    wv[q, b, :] = wrow
                    for (j0, j1, wc) in segs[0]:
                        wsc = wrow[wc]
                        for j in range(j0 // 16, j1 // 16):
                            sl = slice(16 * j, 16 * (j + 1))
                            rows[q, b, sl] = rows[q, b, sl] * wsc
                    return carry2

                lax.fori_loop(0, _B, ebody0, 0, unroll=4)
                if write_w:
                    pltpu.async_copy(wv.at[q],
                                     w_out.at[pl.ds(ebase + nb * _B, _B)],
                                     sem_ww[q])
                pltpu.async_copy(wv.at[q], den_sh.at[idx_d.at[nb]],
                                 sem_dw[q], add=True)
            else:
                if write_w:
                    pltpu.make_async_copy(
                        w_out.at[pl.ds(ebase + nb * _B, _B)],
                        wv.at[q], sem_w[q]).wait()

                def ebody(b, carry2):
                    wrow = wv[q, b, :]
                    for (j0, j1, wc) in segs[ci]:
                        wsc = wrow[wc]
                        for j in range(j0 // 16, j1 // 16):
                            sl = slice(16 * j, 16 * (j + 1))
                            rows[q, b, sl] = rows[q, b, sl] * wsc
                    return carry2

                lax.fori_loop(0, _B, ebody, 0, unroll=4)
            pltpu.async_copy(rows.at[q], acc_sh.at[idx_d.at[nb]],
                             sem_s[q], add=True)

        issue(0, 0, False)
        issue(1, 1, False)
        process(0, 0, True)
        issue(2, 2, False)
        process(1, 1, True)
        issue(3, 0, True)
        process(2, 2, True)
        issue(4, 1, True)

        def lbody(k, carry):
            nb0 = 3 * k
            process(nb0, 0, False)
            issue(nb0 + 2, 2, True)
            process(nb0 + 1, 1, False)
            issue(nb0 + 3, 0, True)
            process(nb0 + 2, 2, False)
            issue(nb0 + 4, 1, True)
            return carry

        lax.fori_loop(1, 41, lbody, 0)
        process(123, 0, False)
        process(124, 1, False)

        for q in range(3):
            pltpu.make_async_copy(rows.at[q], acc_sh.at[idx_d.at[0]],
                                  sem_s[q]).wait()
            if ci == 0:
                pltpu.make_async_copy(wv.at[q], den_sh.at[idx_d.at[0]],
                                      sem_dw[q]).wait()
                if write_w:
                    pltpu.make_async_copy(wv.at[q],
                                          w_out.at[pl.ds(ebase, _B)],
                                          sem_ww[q]).wait()

        plsc.subcore_barrier()
        for k in range(5):
            sz = 128 if k < 4 else _RPS - 4 * 128
            pltpu.sync_copy(acc_sh.at[pl.ds(r0 + k * 128, sz)],
                            acc_out.at[c, ci, pl.ds(r0 + k * 128, sz)])
        if ci == 0:
            pltpu.sync_copy(den_sh.at[pl.ds(r0, _RPS)],
                            den_out.at[c, pl.ds(r0, _RPS)])

        @pl.when(s == 0)
        def _():
            pltpu.sync_copy(acc_sh.at[pl.ds(_NS * _RPS, _TAIL)],
                            acc_out.at[c, ci, pl.ds(_NS * _RPS, _TAIL)])
            if ci == 0:
                pltpu.sync_copy(den_sh.at[pl.ds(_NS * _RPS, _TAIL)],
                                den_out.at[c, pl.ds(_NS * _RPS, _TAIL)])

        if ci + 1 < nchunk:
            plsc.subcore_barrier()


def _fused_call(tabs, segs, D, tab_s, tab_d, src2, dst2, write_w):
    nchunk = len(tabs)
    out_type = [
        jax.ShapeDtypeStruct((_NC, nchunk, _N, D), jnp.float32),
        jax.ShapeDtypeStruct((_NC, _N, 16), jnp.float32),
    ]
    if write_w:
        out_type.append(jax.ShapeDtypeStruct((_E, 16), jnp.float32))

    def body(*refs):
        _fused_body(nchunk, D, segs, write_w, refs)

    fn = pl.kernel(
        body,
        mesh=_mesh(),
        out_type=out_type,
        scratch_types=[
            pltpu.VMEM((_NB, _B), jnp.int32),
            pltpu.VMEM((_NB, _B), jnp.int32),
            pltpu.VMEM((3, _B, D), jnp.float32),
            pltpu.VMEM((3, _B, 16), jnp.float32),
            pltpu.VMEM((3, _B, 16), jnp.float32),
            pltpu.VMEM((3, _B, 16), jnp.float32),
            pltpu.VMEM((128, D), jnp.float32),
            pltpu.VMEM((_RPS, 16), jnp.float32),
            pltpu.VMEM_SHARED((_N, D), jnp.float32),
            pltpu.VMEM_SHARED((_N, 16), jnp.float32),
        ] + [pltpu.SemaphoreType.DMA] * 21,
        compiler_params=_SC_PARAMS,
    )
    return fn(tab_s, tab_d, *tabs, src2, dst2)


def _t2_body(acc_ref, den_ref, b1_ref, w2_ref, a2s_ref, a2d_ref,
             h2_ref, t2s_ref, t2d_ref):
    den = den_ref[0] + den_ref[1]
    blk = den.shape[0]
    h2 = jnp.zeros((blk, _D2), jnp.float32)
    for c8 in range(8):
        a = acc_ref[0, c8] + acc_ref[1, c8]
        denf = den[:, c8:c8 + 1]
        hin = a / (denf + 1e-16) + b1_ref[:, 64 * c8:64 * (c8 + 1)]
        hin = jnp.maximum(hin, 0.0)
        h2 = h2 + jnp.dot(hin, w2_ref[64 * c8:64 * (c8 + 1), :],
                          preferred_element_type=jnp.float32)
    h2_ref[...] = h2
    s2 = jnp.sum(h2 * a2s_ref[...], axis=1, keepdims=True)
    d2 = jnp.sum(h2 * a2d_ref[...], axis=1, keepdims=True)
    t2s_ref[...] = jnp.broadcast_to(s2, (blk, 16))
    t2d_ref[...] = jnp.broadcast_to(d2, (blk, 16))


def _t2_call(acc1, den1, b1r, W2p, a2s, a2d):
    blk = _ROWBLK
    return pl.pallas_call(
        _t2_body,
        grid=(_GRID,),
        in_specs=[
            pl.BlockSpec((_NC, 8, blk, 64), lambda i: (0, 0, i, 0)),
            pl.BlockSpec((_NC, blk, 16), lambda i: (0, i, 0)),
            pl.BlockSpec((1, _HC), lambda i: (0, 0)),
            pl.BlockSpec((_HC, _D2), lambda i: (0, 0)),
            pl.BlockSpec((1, _D2), lambda i: (0, 0)),
            pl.BlockSpec((1, _D2), lambda i: (0, 0)),
        ],
        out_specs=[
            pl.BlockSpec((blk, _D2), lambda i: (i, 0)),
            pl.BlockSpec((blk, 16), lambda i: (i, 0)),
            pl.BlockSpec((blk, 16), lambda i: (i, 0)),
        ],
        out_shape=[
            jax.ShapeDtypeStruct((_N, _D2), jnp.float32),
            jax.ShapeDtypeStruct((_N, 16), jnp.float32),
            jax.ShapeDtypeStruct((_N, 16), jnp.float32),
        ],
    )(acc1, den1, b1r, W2p, a2s, a2d)


def _t3_body(acc_ref, den_ref, b2_ref, out_ref):
    den = den_ref[0, :, 0:1] + den_ref[1, :, 0:1]
    a = acc_ref[0, 0] + acc_ref[1, 0]
    a = a[:, :_NCLS] / (den + 1e-16) + b2_ref[...]
    m = jnp.max(a, axis=1, keepdims=True)
    ex = jnp.exp(a - m)
    out_ref[...] = a - m - jnp.log(jnp.sum(ex, axis=1, keepdims=True))


def _t3_call(acc2, den2, b2r):
    blk = _ROWBLK
    return pl.pallas_call(
        _t3_body,
        grid=(_GRID,),
        in_specs=[
            pl.BlockSpec((_NC, 1, blk, _D2), lambda i: (0, 0, i, 0)),
            pl.BlockSpec((_NC, blk, 16), lambda i: (0, i, 0)),
            pl.BlockSpec((1, _NCLS), lambda i: (0, 0)),
        ],
        out_specs=pl.BlockSpec((blk, _NCLS), lambda i: (i, 0)),
        out_shape=jax.ShapeDtypeStruct((_N, _NCLS), jnp.float32),
    )(acc2, den2, b2r)


def kernel(x, edge_index, W1, att_src1, att_dst1, b1,
           W2, att_src2, att_dst2, b2):
    src2 = edge_index[0].reshape(_NW * _NB, _B)
    dst2 = edge_index[1].reshape(_NW * _NB, _B)

    eye8 = jnp.eye(_HEADS, dtype=jnp.float32)
    a_src = (eye8[:, None, :] * att_src1[:, :, None]).reshape(_HC, _HEADS)
    a_dst = (eye8[:, None, :] * att_dst1[:, :, None]).reshape(_HC, _HEADS)
    A32 = jnp.concatenate([a_src, a_dst, a_dst, a_src], axis=1)

    t1_outs = _t1_call(x, W1, A32)
    h1cs, asad, adas = list(t1_outs[:8]), t1_outs[8], t1_outs[9]

    segs1 = [[(0, 64, c)] for c in range(8)]
    acc1, den1, _ = _fused_call(h1cs, segs1, 64,
                                asad, adas, src2, dst2, write_w=True)

    b1r = b1.reshape(1, _HC)
    W2p = jnp.concatenate(
        [W2, jnp.zeros((_HC, _D2 - _NCLS), jnp.float32)], axis=1)
    a2s = jnp.concatenate(
        [att_src2, jnp.zeros((1, _D2 - _NCLS), jnp.float32)], axis=1)
    a2d = jnp.concatenate(
        [att_dst2, jnp.zeros((1, _D2 - _NCLS), jnp.float32)], axis=1)

    h2pad, t2s, t2d = _t2_call(acc1, den1, b1r, W2p, a2s, a2d)

    acc2, den2 = _fused_call([h2pad], [[(0, _D2, 0)]], _D2,
                             t2s, t2d, src2, dst2, write_w=False)

    b2r = b2.reshape(1, _NCLS)
    return _t3_call(acc2, den2, b2r)


# speedup vs baseline: 37.6083x; 1.0082x over previous
import jax
import jax.numpy as jnp
from jax import lax
from jax.experimental import pallas as pl
from jax.experimental.pallas import tpu as pltpu
from jax.experimental.pallas import tpu_sc as plsc

_N = 10000
_E = 320000
_FIN = 128
_HEADS = 8
_NHID = 64
_HC = _HEADS * _NHID
_NCLS = 40
_D2 = 48

_NC = 2
_NS = 16
_NW = _NC * _NS
_EPW = _E // _NW
_B = 80
_NB = _EPW // _B
_RPS = 624
_TAIL = _N - _NS * _RPS

_ROWBLK = 2000
_GRID = _N // _ROWBLK


def _mesh():
    return plsc.VectorSubcoreMesh(core_axis_name="c", subcore_axis_name="s")


_SC_PARAMS = pltpu.CompilerParams(use_tc_tiling_on_sc=False)


def _t1_body(*refs):
    x_ref, w1_ref, a32_ref = refs[:3]
    houts = refs[3:10]
    asad_ref, adas_ref = refs[10], refs[11]
    h = jnp.dot(x_ref[...], w1_ref[...], preferred_element_type=jnp.float32)
    for c in range(6):
        houts[c][...] = h[:, 80 * c:80 * (c + 1)]
    houts[6][...] = jnp.concatenate(
        [h[:, 480:512], jnp.zeros((h.shape[0], 48), jnp.float32)], axis=1)
    t = jnp.dot(h, a32_ref[...], preferred_element_type=jnp.float32)
    asad_ref[...] = t[:, :16]
    adas_ref[...] = t[:, 16:]


def _t1_call(x, W1, A32):
    blk = _ROWBLK
    return pl.pallas_call(
        _t1_body,
        grid=(_GRID,),
        in_specs=[
            pl.BlockSpec((blk, _FIN), lambda i: (i, 0)),
            pl.BlockSpec((_FIN, _HC), lambda i: (0, 0)),
            pl.BlockSpec((_HC, 32), lambda i: (0, 0)),
        ],
        out_specs=[pl.BlockSpec((blk, 80), lambda i: (i, 0))] * 7
        + [pl.BlockSpec((blk, 16), lambda i: (i, 0))] * 2,
        out_shape=[jax.ShapeDtypeStruct((_N, 80), jnp.float32)] * 7
        + [jax.ShapeDtypeStruct((_N, 16), jnp.float32)] * 2,
    )(x, W1, A32)


def _fused_body(nchunk, D, segs, write_w, refs):
    ntab = nchunk
    tab_s, tab_d = refs[0], refs[1]
    tabs = refs[2:2 + ntab]
    src2, dst2 = refs[2 + ntab], refs[3 + ntab]
    pos = 4 + ntab
    acc_out = refs[pos]; pos += 1
    den_out = refs[pos]; pos += 1
    if write_w:
        w_out = refs[pos]; pos += 1
    (idx_s, idx_d, rows, asv, adv, wv, zb, zden,
     acc_sh, den_sh) = refs[pos:pos + 10]
    sems = refs[pos + 10:]
    sem_h, sem_a, sem_b, sem_w, sem_s, sem_dw, sem_ww = (
        sems[0:3], sems[3:6], sems[6:9], sems[9:12], sems[12:15],
        sems[15:18], sems[18:21])

    c = lax.axis_index("c")
    s = lax.axis_index("s")
    wid = s * _NC + c
    ebase = wid * _EPW
    nd16 = D // 16

    pltpu.sync_copy(src2.at[pl.ds(wid * _NB, _NB)], idx_s)
    pltpu.sync_copy(dst2.at[pl.ds(wid * _NB, _NB)], idx_d)

    def zfill(i, carry):
        for j in range(nd16):
            zb[i, 16 * j:16 * (j + 1)] = jnp.zeros((16,), jnp.float32)
        return carry

    lax.fori_loop(0, 128, zfill, 0)

    def zdfill(i, carry):
        zden[i] = jnp.zeros((16,), jnp.float32)
        return carry

    lax.fori_loop(0, _RPS, zdfill, 0)

    for ci in range(nchunk):
        tab = tabs[ci]
        r0 = s * _RPS
        for k in range(5):
            sz = 128 if k < 4 else _RPS - 4 * 128
            pltpu.sync_copy(zb.at[pl.ds(0, sz)],
                            acc_sh.at[pl.ds(r0 + k * 128, sz)])
        if ci == 0:
            pltpu.sync_copy(zden, den_sh.at[pl.ds(r0, _RPS)])

        @pl.when(s == 0)
        def _():
            pltpu.sync_copy(zb.at[pl.ds(0, _TAIL)],
                            acc_sh.at[pl.ds(_NS * _RPS, _TAIL)])
            if ci == 0:
                pltpu.sync_copy(zden.at[pl.ds(0, _TAIL)],
                                den_sh.at[pl.ds(_NS * _RPS, _TAIL)])

        plsc.subcore_barrier()

        def issue(nb, q, wait_scatter):
            if wait_scatter:
                pltpu.make_async_copy(rows.at[q], acc_sh.at[idx_d.at[0]],
                                      sem_s[q]).wait()
            pltpu.async_copy(tab.at[idx_s.at[nb]], rows.at[q], sem_h[q])
            if ci == 0:
                pltpu.async_copy(tab_s.at[idx_s.at[nb]], asv.at[q], sem_a[q])
                pltpu.async_copy(tab_d.at[idx_d.at[nb]], adv.at[q], sem_b[q])
            elif write_w:
                pltpu.async_copy(w_out.at[pl.ds(ebase + nb * _B, _B)],
                                 wv.at[q], sem_w[q])

        def process(nb, q, first):
            pltpu.make_async_copy(tab.at[idx_s.at[nb]],
                                  rows.at[q], sem_h[q]).wait()
            if ci == 0:
                pltpu.make_async_copy(tab_s.at[idx_s.at[nb]],
                                      asv.at[q], sem_a[q]).wait()
                pltpu.make_async_copy(tab_d.at[idx_d.at[nb]],
                                      adv.at[q], sem_b[q]).wait()
                if not first:
                    pltpu.make_async_copy(wv.at[q], den_sh.at[idx_d.at[0]],
                                          sem_dw[q]).wait()
                    if write_w:
                        pltpu.make_async_copy(
                            wv.at[q], w_out.at[pl.ds(ebase, _B)],
                            sem_ww[q]).wait()

                def ebody0(b, carry2):
                    v = asv[q, b, :] + adv[q, b, :]
                    v = jnp.where(v >= 0.0, v, v * 0.2)
                    wrow = jnp.exp(v)
                    wv[q, b, :] = wrow
                    for (j0, j1, wc) in segs[0]:
                        wsc = wrow[wc]
                        for j in range(j0 // 16, j1 // 16):
                            sl = slice(16 * j, 16 * (j + 1))
                            rows[q, b, sl] = rows[q, b, sl] * wsc
                    return carry2

                lax.fori_loop(0, _B, ebody0, 0)
                if write_w:
                    pltpu.async_copy(wv.at[q],
                                     w_out.at[pl.ds(ebase + nb * _B, _B)],
                                     sem_ww[q])
                pltpu.async_copy(wv.at[q], den_sh.at[idx_d.at[nb]],
                                 sem_dw[q], add=True)
            else:
                if write_w:
                    pltpu.make_async_copy(
                        w_out.at[pl.ds(ebase + nb * _B, _B)],
                        wv.at[q], sem_w[q]).wait()

                def ebody(b, carry2):
                    wrow = wv[q, b, :]
                    for (j0, j1, wc) in segs[ci]:
                        wsc = wrow[wc]
                        for j in range(j0 // 16, j1 // 16):
                            sl = slice(16 * j, 16 * (j + 1))
                            rows[q, b, sl] = rows[q, b, sl] * wsc
                    return carry2

                lax.fori_loop(0, _B, ebody, 0)
            pltpu.async_copy(rows.at[q], acc_sh.at[idx_d.at[nb]],
                             sem_s[q], add=True)

        issue(0, 0, False)
        issue(1, 1, False)
        process(0, 0, True)
        issue(2, 2, False)
        process(1, 1, True)
        issue(3, 0, True)
        process(2, 2, True)
        issue(4, 1, True)

        def lbody(k, carry):
            nb0 = 3 * k
            process(nb0, 0, False)
            issue(nb0 + 2, 2, True)
            process(nb0 + 1, 1, False)
            issue(nb0 + 3, 0, True)
            process(nb0 + 2, 2, False)
            issue(nb0 + 4, 1, True)
            return carry

        lax.fori_loop(1, 41, lbody, 0)
        process(123, 0, False)
        process(124, 1, False)

        for q in range(3):
            pltpu.make_async_copy(rows.at[q], acc_sh.at[idx_d.at[0]],
                                  sem_s[q]).wait()
            if ci == 0:
                pltpu.make_async_copy(wv.at[q], den_sh.at[idx_d.at[0]],
                                      sem_dw[q]).wait()
                if write_w:
                    pltpu.make_async_copy(wv.at[q],
                                          w_out.at[pl.ds(ebase, _B)],
                                          sem_ww[q]).wait()

        plsc.subcore_barrier()
        for k in range(5):
            sz = 128 if k < 4 else _RPS - 4 * 128
            pltpu.sync_copy(acc_sh.at[pl.ds(r0 + k * 128, sz)],
                            acc_out.at[c, ci, pl.ds(r0 + k * 128, sz)])
        if ci == 0:
            pltpu.sync_copy(den_sh.at[pl.ds(r0, _RPS)],
                            den_out.at[c, pl.ds(r0, _RPS)])

        @pl.when(s == 0)
        def _():
            pltpu.sync_copy(acc_sh.at[pl.ds(_NS * _RPS, _TAIL)],
                            acc_out.at[c, ci, pl.ds(_NS * _RPS, _TAIL)])
            if ci == 0:
                pltpu.sync_copy(den_sh.at[pl.ds(_NS * _RPS, _TAIL)],
                                den_out.at[c, pl.ds(_NS * _RPS, _TAIL)])

        if ci + 1 < nchunk:
            plsc.subcore_barrier()


def _fused_call(tabs, segs, D, tab_s, tab_d, src2, dst2, write_w):
    nchunk = len(tabs)
    out_type = [
        jax.ShapeDtypeStruct((_NC, nchunk, _N, D), jnp.float32),
        jax.ShapeDtypeStruct((_NC, _N, 16), jnp.float32),
    ]
    if write_w:
        out_type.append(jax.ShapeDtypeStruct((_E, 16), jnp.float32))

    def body(*refs):
        _fused_body(nchunk, D, segs, write_w, refs)

    fn = pl.kernel(
        body,
        mesh=_mesh(),
        out_type=out_type,
        scratch_types=[
            pltpu.VMEM((_NB, _B), jnp.int32),
            pltpu.VMEM((_NB, _B), jnp.int32),
            pltpu.VMEM((3, _B, D), jnp.float32),
            pltpu.VMEM((3, _B, 16), jnp.float32),
            pltpu.VMEM((3, _B, 16), jnp.float32),
            pltpu.VMEM((3, _B, 16), jnp.float32),
            pltpu.VMEM((128, D), jnp.float32),
            pltpu.VMEM((_RPS, 16), jnp.float32),
            pltpu.VMEM_SHARED((_N, D), jnp.float32),
            pltpu.VMEM_SHARED((_N, 16), jnp.float32),
        ] + [pltpu.SemaphoreType.DMA] * 21,
        compiler_params=_SC_PARAMS,
    )
    return fn(tab_s, tab_d, *tabs, src2, dst2)


def _t2_body(acc_ref, den_ref, b1_ref, w2_ref, a2s_ref, a2d_ref,
             h2_ref, t2s_ref, t2d_ref):
    den = den_ref[0] + den_ref[1]
    blk = den.shape[0]
    hcat = jnp.concatenate(
        [acc_ref[0, ci] + acc_ref[1, ci] for ci in range(7)], axis=1)
    h2 = jnp.zeros((blk, _D2), jnp.float32)
    for hd in range(8):
        a = hcat[:, 64 * hd:64 * (hd + 1)]
        denf = den[:, hd:hd + 1]
        hin = a / (denf + 1e-16) + b1_ref[:, 64 * hd:64 * (hd + 1)]
        hin = jnp.maximum(hin, 0.0)
        h2 = h2 + jnp.dot(hin, w2_ref[64 * hd:64 * (hd + 1), :],
                          preferred_element_type=jnp.float32)
    h2_ref[...] = h2
    s2 = jnp.sum(h2 * a2s_ref[...], axis=1, keepdims=True)
    d2 = jnp.sum(h2 * a2d_ref[...], axis=1, keepdims=True)
    t2s_ref[...] = jnp.broadcast_to(s2, (blk, 16))
    t2d_ref[...] = jnp.broadcast_to(d2, (blk, 16))


def _t2_call(acc1, den1, b1r, W2p, a2s, a2d):
    blk = _ROWBLK
    return pl.pallas_call(
        _t2_body,
        grid=(_GRID,),
        in_specs=[
            pl.BlockSpec((_NC, 7, blk, 80), lambda i: (0, 0, i, 0)),
            pl.BlockSpec((_NC, blk, 16), lambda i: (0, i, 0)),
            pl.BlockSpec((1, _HC), lambda i: (0, 0)),
            pl.BlockSpec((_HC, _D2), lambda i: (0, 0)),
            pl.BlockSpec((1, _D2), lambda i: (0, 0)),
            pl.BlockSpec((1, _D2), lambda i: (0, 0)),
        ],
        out_specs=[
            pl.BlockSpec((blk, _D2), lambda i: (i, 0)),
            pl.BlockSpec((blk, 16), lambda i: (i, 0)),
            pl.BlockSpec((blk, 16), lambda i: (i, 0)),
        ],
        out_shape=[
            jax.ShapeDtypeStruct((_N, _D2), jnp.float32),
            jax.ShapeDtypeStruct((_N, 16), jnp.float32),
            jax.ShapeDtypeStruct((_N, 16), jnp.float32),
        ],
    )(acc1, den1, b1r, W2p, a2s, a2d)


def _t3_body(acc_ref, den_ref, b2_ref, out_ref):
    den = den_ref[0, :, 0:1] + den_ref[1, :, 0:1]
    a = acc_ref[0, 0] + acc_ref[1, 0]
    a = a[:, :_NCLS] / (den + 1e-16) + b2_ref[...]
    m = jnp.max(a, axis=1, keepdims=True)
    ex = jnp.exp(a - m)
    out_ref[...] = a - m - jnp.log(jnp.sum(ex, axis=1, keepdims=True))


def _t3_call(acc2, den2, b2r):
    blk = _ROWBLK
    return pl.pallas_call(
        _t3_body,
        grid=(_GRID,),
        in_specs=[
            pl.BlockSpec((_NC, 1, blk, _D2), lambda i: (0, 0, i, 0)),
            pl.BlockSpec((_NC, blk, 16), lambda i: (0, i, 0)),
            pl.BlockSpec((1, _NCLS), lambda i: (0, 0)),
        ],
        out_specs=pl.BlockSpec((blk, _NCLS), lambda i: (i, 0)),
        out_shape=jax.ShapeDtypeStruct((_N, _NCLS), jnp.float32),
    )(acc2, den2, b2r)


def kernel(x, edge_index, W1, att_src1, att_dst1, b1,
           W2, att_src2, att_dst2, b2):
    src2 = edge_index[0].reshape(_NW * _NB, _B)
    dst2 = edge_index[1].reshape(_NW * _NB, _B)

    eye8 = jnp.eye(_HEADS, dtype=jnp.float32)
    a_src = (eye8[:, None, :] * att_src1[:, :, None]).reshape(_HC, _HEADS)
    a_dst = (eye8[:, None, :] * att_dst1[:, :, None]).reshape(_HC, _HEADS)
    A32 = jnp.concatenate([a_src, a_dst, a_dst, a_src], axis=1)

    t1_outs = _t1_call(x, W1, A32)
    h1cs, asad, adas = list(t1_outs[:7]), t1_outs[7], t1_outs[8]

    segs1 = []
    for ci in range(7):
        lo, hi, f, sg = 80 * ci, 80 * ci + 80, 80 * ci, []
        while f < hi:
            if f >= 512:
                sg.append((f - lo, hi - lo, 0))
                break
            nxt = min((f // 64 + 1) * 64, hi)
            sg.append((f - lo, nxt - lo, f // 64))
            f = nxt
        segs1.append(sg)
    acc1, den1, _ = _fused_call(h1cs, segs1, 80,
                                asad, adas, src2, dst2, write_w=True)

    b1r = b1.reshape(1, _HC)
    W2p = jnp.concatenate(
        [W2, jnp.zeros((_HC, _D2 - _NCLS), jnp.float32)], axis=1)
    a2s = jnp.concatenate(
        [att_src2, jnp.zeros((1, _D2 - _NCLS), jnp.float32)], axis=1)
    a2d = jnp.concatenate(
        [att_dst2, jnp.zeros((1, _D2 - _NCLS), jnp.float32)], axis=1)

    h2pad, t2s, t2d = _t2_call(acc1, den1, b1r, W2p, a2s, a2d)

    acc2, den2 = _fused_call([h2pad], [[(0, _D2, 0)]], _D2,
                             t2s, t2d, src2, dst2, write_w=False)

    b2r = b2.reshape(1, _NCLS)
    return _t3_call(acc2, den2, b2r)
